# Initial kernel scaffold; baseline (speedup 1.0000x reference)
import jax
import jax.numpy as jnp
from jax.experimental import pallas as pl


def kernel(x, y, w):
    raise NotImplementedError("write your pallas kernel here")


# speedup vs baseline: 1.3466x; 1.3466x over previous
import functools

import jax
import jax.numpy as jnp
from jax.experimental import pallas as pl

EPS_ = 1e-08
B_, C_ = 16384, 1000
BLK_ = 1024


def _body(y_ref, x_ref, w_ref, out_ref):
    i = pl.program_id(0)
    y_col = y_ref[0]
    classes = jax.lax.broadcasted_iota(jnp.int32, (BLK_, C_), 1)
    onehot = (y_col == classes).astype(jnp.bfloat16)
    w_b = w_ref[...].astype(jnp.bfloat16)
    wy = jax.lax.dot_general(
        onehot, w_b, (((1,), (0,)), ((), ())),
        preferred_element_type=jnp.float32)
    s = jnp.sum(wy * x_ref[...], axis=1, keepdims=True)
    part = jnp.sum(-jnp.log(s + EPS_)).reshape(1, 1)

    @pl.when(i == 0)
    def _():
        out_ref[...] = jnp.zeros((1, 1), jnp.float32)

    out_ref[...] += part


@jax.jit
def kernel(x, y, w):
    nblk = B_ // BLK_
    y3 = y.astype(jnp.int32).reshape(nblk, BLK_, 1)
    total = pl.pallas_call(
        _body,
        grid=(nblk,),
        in_specs=[
            pl.BlockSpec((1, BLK_, 1), lambda i: (i, 0, 0)),
            pl.BlockSpec((BLK_, C_), lambda i: (i, 0)),
            pl.BlockSpec((C_, C_), lambda i: (0, 0)),
        ],
        out_specs=pl.BlockSpec((1, 1), lambda i: (0, 0)),
        out_shape=jax.ShapeDtypeStruct((1, 1), jnp.float32),
    )(y3, x, w)
    return total[0, 0] / B_
